# Initial kernel scaffold; baseline (speedup 1.0000x reference)
import jax
import jax.numpy as jnp
from jax.experimental import pallas as pl


def kernel(bin_ids, subbin_ids, pos_table, bin_table, subbin_table, cls_table):
    raise NotImplementedError("write your pallas kernel here")


# speedup vs baseline: 7.3055x; 7.3055x over previous
import functools

import jax
import jax.numpy as jnp
from jax import lax
from jax.experimental import pallas as pl
from jax.experimental.pallas import tpu as pltpu
from jax.experimental.pallas import tpu_sc as plsc

B, L, D = 4096, 200, 32
NC, NS = 2, 16
NW = NC * NS
ROWS_PER_W = B // NW
C0, C1 = 104, 96
LANES = 16

_mesh = plsc.VectorSubcoreMesh(
    core_axis_name="c", subcore_axis_name="s", num_cores=NC, num_subcores=NS
)


@functools.partial(
    pl.kernel,
    out_type=jax.ShapeDtypeStruct((B, L + 1, D), jnp.float32),
    mesh=_mesh,
    scratch_types=[
        pltpu.VMEM((ROWS_PER_W * L,), jnp.int32),
        pltpu.VMEM((ROWS_PER_W * L,), jnp.int32),
        pltpu.VMEM((L, D), jnp.float32),
        pltpu.VMEM((L, D), jnp.float32),
        pltpu.VMEM((L, D), jnp.float32),
        pltpu.VMEM((L + 1, D), jnp.float32),
        pltpu.SemaphoreType.DMA,
    ],
    compiler_params=pltpu.CompilerParams(use_tc_tiling_on_sc=False),
)
def _emb_kernel(bin_ids_hbm, subbin_ids_hbm, pos_hbm, bin_tab_hbm,
                subbin_tab_hbm, cls_hbm, out_hbm,
                bin_idx_v, sub_idx_v, bin_rows_v, sub_rows_v, pos_v, acc_v,
                sem):
    wid = lax.axis_index("s") * NC + lax.axis_index("c")
    base = wid * ROWS_PER_W

    pltpu.sync_copy(bin_ids_hbm.at[pl.ds(base * L, ROWS_PER_W * L)], bin_idx_v)
    pltpu.sync_copy(subbin_ids_hbm.at[pl.ds(base * L, ROWS_PER_W * L)],
                    sub_idx_v)
    pltpu.sync_copy(pos_hbm.at[pl.ds(0, L)], pos_v)
    pltpu.sync_copy(cls_hbm, acc_v.at[pl.ds(0, 1)])

    def row_body(r, carry):
        off = r * L
        c0 = pltpu.async_copy(
            bin_tab_hbm.at[bin_idx_v.at[pl.ds(off, C0)]],
            bin_rows_v.at[pl.ds(0, C0)], sem)
        c1 = pltpu.async_copy(
            bin_tab_hbm.at[bin_idx_v.at[pl.ds(off + C0, C1)]],
            bin_rows_v.at[pl.ds(C0, C1)], sem)
        c2 = pltpu.async_copy(
            subbin_tab_hbm.at[sub_idx_v.at[pl.ds(off, C0)]],
            sub_rows_v.at[pl.ds(0, C0)], sem)
        c3 = pltpu.async_copy(
            subbin_tab_hbm.at[sub_idx_v.at[pl.ds(off + C0, C1)]],
            sub_rows_v.at[pl.ds(C0, C1)], sem)
        c0.wait()
        c1.wait()
        c2.wait()
        c3.wait()

        def add_body(j, carry2):
            for h in (0, LANES):
                acc_v[j + 1, pl.ds(h, LANES)] = (
                    bin_rows_v[j, pl.ds(h, LANES)]
                    + sub_rows_v[j, pl.ds(h, LANES)]
                    + pos_v[j, pl.ds(h, LANES)]
                )
            return carry2

        lax.fori_loop(0, L, add_body, 0)
        pltpu.sync_copy(acc_v, out_hbm.at[base + r])
        return carry

    lax.fori_loop(0, ROWS_PER_W, row_body, 0)


def kernel(bin_ids, subbin_ids, pos_table, bin_table, subbin_table, cls_table):
    return _emb_kernel(bin_ids.astype(jnp.int32).reshape(-1),
                       subbin_ids.astype(jnp.int32).reshape(-1),
                       pos_table, bin_table, subbin_table, cls_table)


# speedup vs baseline: 9.0098x; 1.2333x over previous
import functools

import jax
import jax.numpy as jnp
from jax import lax
from jax.experimental import pallas as pl
from jax.experimental.pallas import tpu as pltpu
from jax.experimental.pallas import tpu_sc as plsc

B, L, D = 4096, 200, 32
NC, NS = 2, 16
NW = NC * NS
ROWS_PER_W = B // NW
C0, C1 = 104, 96
LANES = 16

_mesh = plsc.VectorSubcoreMesh(
    core_axis_name="c", subcore_axis_name="s", num_cores=NC, num_subcores=NS
)


@functools.partial(
    pl.kernel,
    out_type=jax.ShapeDtypeStruct((B, L + 1, D), jnp.float32),
    mesh=_mesh,
    scratch_types=[
        pltpu.VMEM((ROWS_PER_W * L,), jnp.int32),
        pltpu.VMEM((ROWS_PER_W * L,), jnp.int32),
        pltpu.VMEM((L, D), jnp.float32),
        pltpu.VMEM((L, D), jnp.float32),
        pltpu.VMEM((L, D), jnp.float32),
        pltpu.VMEM((L, D), jnp.float32),
        pltpu.VMEM((L, D), jnp.float32),
        pltpu.VMEM((L + 1, D), jnp.float32),
        pltpu.VMEM((L + 1, D), jnp.float32),
        pltpu.SemaphoreType.DMA,
        pltpu.SemaphoreType.DMA,
        pltpu.SemaphoreType.DMA,
        pltpu.SemaphoreType.DMA,
    ],
    compiler_params=pltpu.CompilerParams(use_tc_tiling_on_sc=False),
)
def _emb_kernel(bin_ids_hbm, subbin_ids_hbm, pos_hbm, bin_tab_hbm,
                subbin_tab_hbm, cls_hbm, out_hbm,
                bin_idx_v, sub_idx_v, bin_a, sub_a, bin_b, sub_b, pos_v,
                acc_a, acc_b, sem_a, sem_b, sem_oa, sem_ob):
    wid = lax.axis_index("s") * NC + lax.axis_index("c")
    base = wid * ROWS_PER_W

    pltpu.sync_copy(bin_ids_hbm.at[pl.ds(base * L, ROWS_PER_W * L)], bin_idx_v)
    pltpu.sync_copy(subbin_ids_hbm.at[pl.ds(base * L, ROWS_PER_W * L)],
                    sub_idx_v)
    pltpu.sync_copy(pos_hbm.at[pl.ds(0, L)], pos_v)
    pltpu.sync_copy(cls_hbm, acc_a.at[pl.ds(0, 1)])
    pltpu.sync_copy(cls_hbm, acc_b.at[pl.ds(0, 1)])

    def fire_gather(r, bin_v, sub_v, sem):
        off = r * L
        pltpu.async_copy(bin_tab_hbm.at[bin_idx_v.at[pl.ds(off, C0)]],
                         bin_v.at[pl.ds(0, C0)], sem)
        pltpu.async_copy(bin_tab_hbm.at[bin_idx_v.at[pl.ds(off + C0, C1)]],
                         bin_v.at[pl.ds(C0, C1)], sem)
        pltpu.async_copy(subbin_tab_hbm.at[sub_idx_v.at[pl.ds(off, C0)]],
                         sub_v.at[pl.ds(0, C0)], sem)
        pltpu.async_copy(subbin_tab_hbm.at[sub_idx_v.at[pl.ds(off + C0, C1)]],
                         sub_v.at[pl.ds(C0, C1)], sem)

    def drain_gather(bin_v, sub_v, sem):
        pltpu.make_async_copy(bin_tab_hbm.at[pl.ds(0, L)], bin_v, sem).wait()
        pltpu.make_async_copy(subbin_tab_hbm.at[pl.ds(0, L)], sub_v,
                              sem).wait()

    def drain_out(acc_v, sem):
        pltpu.make_async_copy(acc_v, out_hbm.at[base], sem).wait()

    def compute(bin_v, sub_v, acc_v):
        @plsc.parallel_loop(0, L, step=1, unroll=8)
        def _(j):
            for h in (0, LANES):
                acc_v[j + 1, pl.ds(h, LANES)] = (
                    bin_v[j, pl.ds(h, LANES)]
                    + sub_v[j, pl.ds(h, LANES)]
                    + pos_v[j, pl.ds(h, LANES)]
                )

    fire_gather(0, bin_a, sub_a, sem_a)

    def pair_body(g, carry):
        r0 = 2 * g
        fire_gather(r0 + 1, bin_b, sub_b, sem_b)
        drain_gather(bin_a, sub_a, sem_a)

        @pl.when(g > 0)
        def _():
            drain_out(acc_a, sem_oa)

        compute(bin_a, sub_a, acc_a)
        pltpu.async_copy(acc_a, out_hbm.at[base + r0], sem_oa)

        @pl.when(g < ROWS_PER_W // 2 - 1)
        def _():
            fire_gather(r0 + 2, bin_a, sub_a, sem_a)

        drain_gather(bin_b, sub_b, sem_b)

        @pl.when(g > 0)
        def _():
            drain_out(acc_b, sem_ob)

        compute(bin_b, sub_b, acc_b)
        pltpu.async_copy(acc_b, out_hbm.at[base + r0 + 1], sem_ob)
        return carry

    lax.fori_loop(0, ROWS_PER_W // 2, pair_body, 0)
    drain_out(acc_a, sem_oa)
    drain_out(acc_b, sem_ob)


def kernel(bin_ids, subbin_ids, pos_table, bin_table, subbin_table, cls_table):
    return _emb_kernel(bin_ids.astype(jnp.int32).reshape(-1),
                       subbin_ids.astype(jnp.int32).reshape(-1),
                       pos_table, bin_table, subbin_table, cls_table)


# speedup vs baseline: 9.0142x; 1.0005x over previous
import functools

import jax
import jax.numpy as jnp
from jax import lax
from jax.experimental import pallas as pl
from jax.experimental.pallas import tpu as pltpu
from jax.experimental.pallas import tpu_sc as plsc

B, L, D = 4096, 200, 32
NC, NS = 2, 16
NW = NC * NS
ROWS_PER_W = B // NW
C0, C1 = 104, 96
LANES = 16
BLK = (L + 1) * D

_mesh = plsc.VectorSubcoreMesh(
    core_axis_name="c", subcore_axis_name="s", num_cores=NC, num_subcores=NS
)


@functools.partial(
    pl.kernel,
    out_type=jax.ShapeDtypeStruct((B * BLK,), jnp.float32),
    mesh=_mesh,
    scratch_types=[
        pltpu.VMEM((ROWS_PER_W * L,), jnp.int32),
        pltpu.VMEM((ROWS_PER_W * L,), jnp.int32),
        pltpu.VMEM((L, D), jnp.float32),
        pltpu.VMEM((L, D), jnp.float32),
        pltpu.VMEM((L, D), jnp.float32),
        pltpu.VMEM((L, D), jnp.float32),
        pltpu.VMEM((L, D), jnp.float32),
        pltpu.VMEM((BLK,), jnp.float32),
        pltpu.VMEM((BLK,), jnp.float32),
        pltpu.SemaphoreType.DMA,
        pltpu.SemaphoreType.DMA,
        pltpu.SemaphoreType.DMA,
        pltpu.SemaphoreType.DMA,
    ],
    compiler_params=pltpu.CompilerParams(use_tc_tiling_on_sc=False),
)
def _emb_kernel(bin_ids_hbm, subbin_ids_hbm, pos_hbm, bin_tab_hbm,
                subbin_tab_hbm, cls_hbm, out_hbm,
                bin_idx_v, sub_idx_v, bin_a, sub_a, bin_b, sub_b, pos_v,
                acc_a, acc_b, sem_a, sem_b, sem_oa, sem_ob):
    wid = lax.axis_index("s") * NC + lax.axis_index("c")
    base = wid * ROWS_PER_W

    pltpu.sync_copy(bin_ids_hbm.at[pl.ds(base * L, ROWS_PER_W * L)], bin_idx_v)
    pltpu.sync_copy(subbin_ids_hbm.at[pl.ds(base * L, ROWS_PER_W * L)],
                    sub_idx_v)
    pltpu.sync_copy(pos_hbm.at[pl.ds(0, L)], pos_v)
    pltpu.sync_copy(cls_hbm, acc_a.at[pl.ds(0, D)])
    pltpu.sync_copy(cls_hbm, acc_b.at[pl.ds(0, D)])

    def fire_gather(r, bin_v, sub_v, sem):
        off = r * L
        pltpu.async_copy(bin_tab_hbm.at[bin_idx_v.at[pl.ds(off, C0)]],
                         bin_v.at[pl.ds(0, C0)], sem)
        pltpu.async_copy(bin_tab_hbm.at[bin_idx_v.at[pl.ds(off + C0, C1)]],
                         bin_v.at[pl.ds(C0, C1)], sem)
        pltpu.async_copy(subbin_tab_hbm.at[sub_idx_v.at[pl.ds(off, C0)]],
                         sub_v.at[pl.ds(0, C0)], sem)
        pltpu.async_copy(subbin_tab_hbm.at[sub_idx_v.at[pl.ds(off + C0, C1)]],
                         sub_v.at[pl.ds(C0, C1)], sem)

    def drain_gather(bin_v, sub_v, sem):
        pltpu.make_async_copy(bin_tab_hbm.at[pl.ds(0, L)], bin_v, sem).wait()
        pltpu.make_async_copy(subbin_tab_hbm.at[pl.ds(0, L)], sub_v,
                              sem).wait()

    def drain_out(acc_v, sem):
        pltpu.make_async_copy(acc_v, out_hbm.at[pl.ds(0, BLK)], sem).wait()

    def compute(bin_v, sub_v, acc_v):
        @plsc.parallel_loop(0, L, step=1, unroll=8)
        def _(j):
            for h in (0, LANES):
                acc_v[pl.ds(D + j * D + h, LANES)] = (
                    bin_v[j, pl.ds(h, LANES)]
                    + sub_v[j, pl.ds(h, LANES)]
                    + pos_v[j, pl.ds(h, LANES)]
                )

    fire_gather(0, bin_a, sub_a, sem_a)

    def pair_body(g, carry):
        r0 = 2 * g
        fire_gather(r0 + 1, bin_b, sub_b, sem_b)
        drain_gather(bin_a, sub_a, sem_a)

        @pl.when(g > 0)
        def _():
            drain_out(acc_a, sem_oa)

        compute(bin_a, sub_a, acc_a)
        pltpu.async_copy(acc_a, out_hbm.at[pl.ds((base + r0) * BLK, BLK)],
                         sem_oa)

        @pl.when(g < ROWS_PER_W // 2 - 1)
        def _():
            fire_gather(r0 + 2, bin_a, sub_a, sem_a)

        drain_gather(bin_b, sub_b, sem_b)

        @pl.when(g > 0)
        def _():
            drain_out(acc_b, sem_ob)

        compute(bin_b, sub_b, acc_b)
        pltpu.async_copy(acc_b, out_hbm.at[pl.ds((base + r0 + 1) * BLK, BLK)],
                         sem_ob)
        return carry

    lax.fori_loop(0, ROWS_PER_W // 2, pair_body, 0)
    drain_out(acc_a, sem_oa)
    drain_out(acc_b, sem_ob)


def kernel(bin_ids, subbin_ids, pos_table, bin_table, subbin_table, cls_table):
    flat = _emb_kernel(bin_ids.astype(jnp.int32).reshape(-1),
                       subbin_ids.astype(jnp.int32).reshape(-1),
                       pos_table, bin_table, subbin_table,
                       cls_table.reshape(-1))
    return flat.reshape(B, L + 1, D)
